# Initial kernel scaffold; baseline (speedup 1.0000x reference)
import jax
import jax.numpy as jnp
from jax.experimental import pallas as pl


def kernel(nodes, neig_nodes, feats, W_self, W_neigh):
    raise NotImplementedError("write your pallas kernel here")


# speedup vs baseline: 1.3676x; 1.3676x over previous
import functools

import jax
import jax.numpy as jnp
from jax import lax
from jax.experimental import pallas as pl
from jax.experimental.pallas import tpu as pltpu
from jax.experimental.pallas import tpu_sc as plsc

D = 128
K = 16
NC, NS = 2, 16
NW = NC * NS
SB = 8


def _sc_gather_body(nodes_hbm, neig_hbm, feats_hbm, self_out, sum_out,
                    sidx_v, nidx_v, srows_v, nrows_v, acc_v, sem, sem2):
    bpw = self_out.shape[0] // NW
    steps = bpw // SB
    wid = lax.axis_index("s") * NC + lax.axis_index("c")
    base = wid * bpw

    pltpu.sync_copy(nodes_hbm.at[pl.ds(base, bpw)], sidx_v)
    pltpu.sync_copy(neig_hbm.at[pl.ds(base * K, bpw * K)], nidx_v)

    def step(k, carry):
        b0 = base + k * SB
        cpn = pltpu.async_copy(
            feats_hbm.at[nidx_v.at[pl.ds(k * (SB * K), SB * K)]], nrows_v, sem)
        cps = pltpu.async_copy(
            feats_hbm.at[sidx_v.at[pl.ds(k * SB, SB)]], srows_v, sem2)
        cpn.wait()
        def node(n, c2):
            for c in range(D // 16):
                s = nrows_v[n * K, pl.ds(c * 16, 16)]
                for j in range(1, K):
                    s = s + nrows_v[n * K + j, pl.ds(c * 16, 16)]
                acc_v[n, pl.ds(c * 16, 16)] = s
            return c2
        lax.fori_loop(0, SB, node, 0, unroll=False)
        cps.wait()
        pltpu.sync_copy(acc_v, sum_out.at[pl.ds(b0, SB)])
        pltpu.sync_copy(srows_v, self_out.at[pl.ds(b0, SB)])
        return carry

    lax.fori_loop(0, steps, step, 0, unroll=False)


def _make_sc_gather(b_pad):
    bpw = b_pad // NW
    mesh = plsc.VectorSubcoreMesh(core_axis_name="c", subcore_axis_name="s")
    return pl.kernel(
        _sc_gather_body,
        out_type=[
            jax.ShapeDtypeStruct((b_pad, D), jnp.float32),
            jax.ShapeDtypeStruct((b_pad, D), jnp.float32),
        ],
        mesh=mesh,
        scratch_types=[
            pltpu.VMEM((bpw,), jnp.int32),
            pltpu.VMEM((bpw * K,), jnp.int32),
            pltpu.VMEM((SB, D), jnp.float32),
            pltpu.VMEM((SB * K, D), jnp.float32),
            pltpu.VMEM((SB, D), jnp.float32),
            pltpu.SemaphoreType.DMA,
            pltpu.SemaphoreType.DMA,
        ],
    )


def _mm_body(self_ref, sum_ref, ws_ref, wn_ref, out_ref):
    acc = jnp.dot(self_ref[...], ws_ref[...],
                  preferred_element_type=jnp.float32)
    acc += jnp.dot(sum_ref[...], wn_ref[...] * (1.0 / K),
                   preferred_element_type=jnp.float32)
    out_ref[...] = jnp.maximum(acc, 0.0)


def _mm(self_feats, neigh_sum, w_self, w_neigh, bm):
    b = self_feats.shape[0]
    grid = (b // bm,)
    return pl.pallas_call(
        _mm_body,
        grid=grid,
        in_specs=[
            pl.BlockSpec((bm, D), lambda i: (i, 0)),
            pl.BlockSpec((bm, D), lambda i: (i, 0)),
            pl.BlockSpec((D, D), lambda i: (0, 0)),
            pl.BlockSpec((D, D), lambda i: (0, 0)),
        ],
        out_specs=pl.BlockSpec((bm, D), lambda i: (i, 0)),
        out_shape=jax.ShapeDtypeStruct((b, D), jnp.float32),
    )(self_feats, neigh_sum, w_self, w_neigh)


@jax.jit
def kernel(nodes, neig_nodes, feats, W_self, W_neigh):
    b = nodes.shape[0]
    bm = 2048
    b_pad = -(-b // bm) * bm
    nodes_p = jnp.zeros((b_pad,), jnp.int32).at[:b].set(nodes.astype(jnp.int32))
    neig_p = jnp.zeros((b_pad * K,), jnp.int32).at[:b * K].set(
        neig_nodes.reshape(-1).astype(jnp.int32))
    self_feats, neigh_sum = _make_sc_gather(b_pad)(nodes_p, neig_p, feats)
    out = _mm(self_feats, neigh_sum, W_self, W_neigh, bm=bm)
    return out[:b]


# speedup vs baseline: 1.7237x; 1.2604x over previous
import jax
import jax.numpy as jnp
from jax import lax
from jax.experimental import pallas as pl
from jax.experimental.pallas import tpu as pltpu
from jax.experimental.pallas import tpu_sc as plsc

D = 128
K = 16
NC, NS = 2, 16
NW = NC * NS
SB = 8
IDX = SB * K
NBUF = 4
SPG = 8
GN = SB * SPG


def _sc_gather_body(nodes_hbm, neig_hbm, feats_hbm, self_out, sum_out,
                    sidx_v, nidx_v, nr0, nr1, nr2, nr3, sr0, sr1, ac0, ac1,
                    sg0, sg1, sg2, sg3, ss0, ss1, st0, st1):
    nrows = (nr0, nr1, nr2, nr3)
    srows = (sr0, sr1)
    accs = (ac0, ac1)
    sgs = (sg0, sg1, sg2, sg3)
    sss = (ss0, ss1)
    sts = (st0, st1)
    bpw = self_out.shape[0] // NW
    steps = bpw // SB
    groups = bpw // GN
    pairs = groups // 2
    last = steps - 1
    wid = lax.axis_index("s") * NC + lax.axis_index("c")
    base = wid * bpw

    pltpu.sync_copy(nodes_hbm.at[pl.ds(base, bpw)], sidx_v)
    pltpu.sync_copy(neig_hbm.at[pl.ds(base * K, bpw * K)], nidx_v)

    def fire_neigh(k, slot):
        kk = jnp.minimum(k, last)
        pltpu.make_async_copy(
            feats_hbm.at[nidx_v.at[pl.ds(kk * IDX, IDX)]],
            nrows[slot], sgs[slot]).start()

    def wait_neigh(slot):
        pltpu.make_async_copy(
            feats_hbm.at[nidx_v.at[pl.ds(0, IDX)]],
            nrows[slot], sgs[slot]).wait()

    def fire_self(g, p):
        pltpu.make_async_copy(
            feats_hbm.at[sidx_v.at[pl.ds(g * GN, GN)]],
            srows[p], sss[p]).start()

    def wait_self(p):
        pltpu.make_async_copy(
            feats_hbm.at[sidx_v.at[pl.ds(0, GN)]],
            srows[p], sss[p]).wait()

    def fire_store(g, p):
        gb = base + g * GN
        pltpu.make_async_copy(accs[p], sum_out.at[pl.ds(gb, GN)],
                              sts[p]).start()
        pltpu.make_async_copy(srows[p], self_out.at[pl.ds(gb, GN)],
                              sts[p]).start()

    def wait_store(p):
        pltpu.make_async_copy(accs[p], sum_out.at[pl.ds(base, GN)],
                              sts[p]).wait()
        pltpu.make_async_copy(srows[p], self_out.at[pl.ds(base, GN)],
                              sts[p]).wait()

    def compute_step(s, slot, p):
        nb = nrows[slot]
        ab = accs[p]

        def node(n, c0):
            def chunk(c, c1):
                v = nb[n * K, pl.ds(c * 16, 16)]
                for j in range(1, K):
                    v = v + nb[n * K + j, pl.ds(c * 16, 16)]
                ab[s * SB + n, pl.ds(c * 16, 16)] = v
                return c1
            return lax.fori_loop(0, D // 16, chunk, c0, unroll=False)
        lax.fori_loop(0, SB, node, 0, unroll=False)

    def run_group(g, p):
        for s in range(SPG):
            slot = s % NBUF
            wait_neigh(slot)
            compute_step(s, slot, p)
            fire_neigh(g * SPG + s + NBUF, slot)
        wait_self(p)
        fire_store(g, p)

    for kk in range(NBUF):
        fire_neigh(kk, kk)
    fire_self(0, 0)
    fire_self(1, 1)

    for u in range(2):
        run_group(u, u)

    def pair_body(t, carry):
        for u in range(2):
            g = 2 * t + u
            wait_store(u)
            fire_self(g, u)
            run_group(g, u)
        return carry
    lax.fori_loop(1, pairs, pair_body, 0, unroll=False)

    for slot in range(NBUF):
        wait_neigh(slot)
    wait_store(0)
    wait_store(1)


def _make_sc_gather(b_pad):
    bpw = b_pad // NW
    mesh = plsc.VectorSubcoreMesh(core_axis_name="c", subcore_axis_name="s")
    return pl.kernel(
        _sc_gather_body,
        out_type=[
            jax.ShapeDtypeStruct((b_pad, D), jnp.float32),
            jax.ShapeDtypeStruct((b_pad, D), jnp.float32),
        ],
        mesh=mesh,
        scratch_types=[
            pltpu.VMEM((bpw,), jnp.int32),
            pltpu.VMEM((bpw * K,), jnp.int32),
            pltpu.VMEM((IDX, D), jnp.float32),
            pltpu.VMEM((IDX, D), jnp.float32),
            pltpu.VMEM((IDX, D), jnp.float32),
            pltpu.VMEM((IDX, D), jnp.float32),
            pltpu.VMEM((GN, D), jnp.float32),
            pltpu.VMEM((GN, D), jnp.float32),
            pltpu.VMEM((GN, D), jnp.float32),
            pltpu.VMEM((GN, D), jnp.float32),
            pltpu.SemaphoreType.DMA,
            pltpu.SemaphoreType.DMA,
            pltpu.SemaphoreType.DMA,
            pltpu.SemaphoreType.DMA,
            pltpu.SemaphoreType.DMA,
            pltpu.SemaphoreType.DMA,
            pltpu.SemaphoreType.DMA,
            pltpu.SemaphoreType.DMA,
        ],
    )


def _mm_body(self_ref, sum_ref, ws_ref, wn_ref, out_ref):
    acc = jnp.dot(self_ref[...], ws_ref[...],
                  preferred_element_type=jnp.float32)
    acc += jnp.dot(sum_ref[...], wn_ref[...] * (1.0 / K),
                   preferred_element_type=jnp.float32)
    out_ref[...] = jnp.maximum(acc, 0.0)


def _mm(self_feats, neigh_sum, w_self, w_neigh, bm):
    b = self_feats.shape[0]
    return pl.pallas_call(
        _mm_body,
        grid=(b // bm,),
        in_specs=[
            pl.BlockSpec((bm, D), lambda i: (i, 0)),
            pl.BlockSpec((bm, D), lambda i: (i, 0)),
            pl.BlockSpec((D, D), lambda i: (0, 0)),
            pl.BlockSpec((D, D), lambda i: (0, 0)),
        ],
        out_specs=pl.BlockSpec((bm, D), lambda i: (i, 0)),
        out_shape=jax.ShapeDtypeStruct((b, D), jnp.float32),
    )(self_feats, neigh_sum, w_self, w_neigh)


@jax.jit
def kernel(nodes, neig_nodes, feats, W_self, W_neigh):
    b = nodes.shape[0]
    bm = 2048
    b_pad = -(-b // bm) * bm
    nodes_p = jnp.zeros((b_pad,), jnp.int32).at[:b].set(nodes.astype(jnp.int32))
    neig_p = jnp.zeros((b_pad * K,), jnp.int32).at[:b * K].set(
        neig_nodes.reshape(-1).astype(jnp.int32))
    self_feats, neigh_sum = _make_sc_gather(b_pad)(nodes_p, neig_p, feats)
    out = _mm(self_feats, neigh_sum, W_self, W_neigh, bm=bm)
    return out[:b]
